# baseline (speedup 1.0000x reference)
import functools

import jax
import jax.numpy as jnp
from jax import lax
from jax.experimental import pallas as pl
from jax.experimental.pallas import tpu as pltpu
from jax.experimental.pallas import tpu_sc as plsc

B = 16384
D = 128
NUM_REL = 1000

_info = plsc.get_sparse_core_info()
NC, NS, L = _info.num_cores, _info.num_subcores, _info.num_lanes
NW = NC * NS
B_PER_W = B // NW
CHUNK = 64
N_CHUNKS = B_PER_W // CHUNK
GROUPS = CHUNK // L
DSL = D // L
SC_ROWS = NS * CHUNK


def _sc_kernel():
    mesh = plsc.VectorSubcoreMesh(core_axis_name="c", subcore_axis_name="s")

    @functools.partial(
        pl.kernel,
        mesh=mesh,
        out_type=jax.ShapeDtypeStruct((B,), jnp.float32),
        scratch_types=[
            pltpu.VMEM((B_PER_W,), jnp.int32),
            pltpu.VMEM((2, CHUNK, D), jnp.float32),
            pltpu.VMEM((2, CHUNK, D), jnp.float32),
            pltpu.VMEM((2, CHUNK, D), jnp.float32),
            pltpu.VMEM((2, CHUNK), jnp.float32),
            pltpu.VMEM_SHARED((NUM_REL, D), jnp.float32),
            pltpu.VMEM_SHARED((2, SC_ROWS, D), jnp.float32),
            pltpu.VMEM_SHARED((2, SC_ROWS, D), jnp.float32),
        ] + [pltpu.SemaphoreType.DMA] * 11,
    )
    def k(src_hbm, ids_hbm, dst_hbm, table_hbm, out_hbm,
          idx_v, rel_v, src_v, dst_v, out_v, table_sh, src_sh, dst_sh,
          *sems):
        wid = lax.axis_index("s") * NC + lax.axis_index("c")
        sid = lax.axis_index("s")
        base = wid * B_PER_W
        lanes = lax.iota(jnp.int32, L)
        asems = [(sems[0], sems[1]), (sems[2], sems[3])]
        bsems = [(sems[4], sems[5]), (sems[6], sems[7])]
        gsems = [sems[8], sems[9]]
        osem = sems[10]

        dnums = lax.GatherDimensionNumbers(
            offset_dims=(), collapsed_slice_dims=(0,), start_index_map=(0,))

        def lane_perm(x, perm):
            return lax.gather(
                x, perm[:, None], dimension_numbers=dnums, slice_sizes=(1,),
                mode=lax.GatherScatterMode.PROMISE_IN_BOUNDS)

        def lane_sum(x):
            for m in (8, 4, 2, 1):
                x = x + lane_perm(x, jnp.bitwise_xor(lanes, m))
            return x

        my_slab = sid * CHUNK

        def fire_hbm(c):
            bb = c % 2
            sa, da = asems[bb]
            rb = base + c * CHUNK
            return (
                pltpu.async_copy(
                    src_hbm.at[pl.ds(rb, CHUNK)],
                    src_sh.at[bb].at[pl.ds(my_slab, CHUNK)], sa),
                pltpu.async_copy(
                    dst_hbm.at[pl.ds(rb, CHUNK)],
                    dst_sh.at[bb].at[pl.ds(my_slab, CHUNK)], da),
            )

        def fire_xbar(c):
            bb = c % 2
            sb, db = bsems[bb]
            return (
                pltpu.async_copy(
                    src_sh.at[bb].at[pl.ds(my_slab, CHUNK)],
                    src_v.at[bb], sb),
                pltpu.async_copy(
                    dst_sh.at[bb].at[pl.ds(my_slab, CHUNK)],
                    dst_v.at[bb], db),
            )

        def fire_gather(c):
            bb = c % 2
            return pltpu.async_copy(
                table_sh.at[idx_v.at[pl.ds(c * CHUNK, CHUNK)]],
                rel_v.at[bb], gsems[bb])

        ah = {0: fire_hbm(0), 1: fire_hbm(1)}
        pltpu.sync_copy(ids_hbm.at[pl.ds(base, B_PER_W)], idx_v)

        @pl.when(sid < 15)
        def _():
            rslab = sid * 64
            pltpu.sync_copy(table_hbm.at[pl.ds(rslab, 64)],
                            table_sh.at[pl.ds(rslab, 64)])

        @pl.when(sid == 15)
        def _():
            pltpu.sync_copy(table_hbm.at[pl.ds(960, 40)],
                            table_sh.at[pl.ds(960, 40)])

        plsc.subcore_barrier()
        g = {0: fire_gather(0)}
        for h in ah.pop(0):
            h.wait()
        bh = {0: fire_xbar(0)}

        for c in range(N_CHUNKS):
            bb = c % 2
            for h in bh.pop(c):
                h.wait()
            if c + 2 < N_CHUNKS:
                ah[c + 2] = fire_hbm(c + 2)
            if c + 1 < N_CHUNKS:
                for h in ah.pop(c + 1):
                    h.wait()
                bh[c + 1] = fire_xbar(c + 1)
                g[c + 1] = fire_gather(c + 1)
            g.pop(c).wait()

            def group_body(gi, _):
                r0 = gi * L

                def row_body(i, res):
                    r = r0 + i
                    acc0 = acc1 = None
                    for j in range(DSL):
                        sl = pl.ds(j * L, L)
                        p = (src_v[bb, r, sl]
                             * rel_v[bb, r, sl]
                             * dst_v[bb, r, sl])
                        if j % 2 == 0:
                            acc0 = p if acc0 is None else acc0 + p
                        else:
                            acc1 = p if acc1 is None else acc1 + p
                    tot = lane_sum(acc0 + acc1)
                    return jnp.where(lanes == i, tot, res)

                res = lax.fori_loop(0, L, row_body,
                                    jnp.zeros((L,), jnp.float32))
                out_v[bb, pl.ds(r0, L)] = res
                return 0

            lax.fori_loop(0, GROUPS, group_body, 0)
            pltpu.sync_copy(out_v.at[bb],
                            out_hbm.at[pl.ds(base + c * CHUNK, CHUNK)])

    return k


_scorer = _sc_kernel()


@jax.jit
def kernel(src_emb, rel_ids, dst_emb, rel_emb_table):
    ids = rel_ids.astype(jnp.int32)
    return _scorer(src_emb, ids, dst_emb, rel_emb_table)


# speedup vs baseline: 1.1305x; 1.1305x over previous
import functools

import jax
import jax.numpy as jnp
from jax import lax
from jax.experimental import pallas as pl
from jax.experimental.pallas import tpu as pltpu
from jax.experimental.pallas import tpu_sc as plsc

B = 16384
D = 128
NUM_REL = 1000

_info = plsc.get_sparse_core_info()
NC, NS, L = _info.num_cores, _info.num_subcores, _info.num_lanes
NW = NC * NS
B_PER_W = B // NW
CHUNK = 128
N_CHUNKS = B_PER_W // CHUNK
GROUPS = CHUNK // L
DSL = D // L


def _sc_kernel():
    mesh = plsc.VectorSubcoreMesh(core_axis_name="c", subcore_axis_name="s")

    @functools.partial(
        pl.kernel,
        mesh=mesh,
        out_type=jax.ShapeDtypeStruct((B,), jnp.float32),
        scratch_types=[
            pltpu.VMEM((B_PER_W,), jnp.int32),
            pltpu.VMEM((2, CHUNK, D), jnp.float32),
            pltpu.VMEM((2, CHUNK, D), jnp.float32),
            pltpu.VMEM((2, CHUNK, D), jnp.float32),
            pltpu.VMEM((2, CHUNK), jnp.float32),
            pltpu.VMEM_SHARED((NUM_REL, D), jnp.float32),
            pltpu.SemaphoreType.DMA,
            pltpu.SemaphoreType.DMA,
            pltpu.SemaphoreType.DMA,
            pltpu.SemaphoreType.DMA,
            pltpu.SemaphoreType.DMA,
            pltpu.SemaphoreType.DMA,
        ],
    )
    def k(src_hbm, ids_hbm, dst_hbm, table_hbm, out_hbm,
          idx_v, rel_v, src_v, dst_v, out_v, table_sh,
          gs0, ss0, ds0, gs1, ss1, ds1):
        wid = lax.axis_index("s") * NC + lax.axis_index("c")
        sid = lax.axis_index("s")
        base = wid * B_PER_W
        lanes = lax.iota(jnp.int32, L)
        sems = [(gs0, ss0, ds0), (gs1, ss1, ds1)]

        dnums = lax.GatherDimensionNumbers(
            offset_dims=(), collapsed_slice_dims=(0,), start_index_map=(0,))

        def lane_perm(x, perm):
            return lax.gather(
                x, perm[:, None], dimension_numbers=dnums, slice_sizes=(1,),
                mode=lax.GatherScatterMode.PROMISE_IN_BOUNDS)

        def lane_sum(x):
            for m in (8, 4, 2, 1):
                x = x + lane_perm(x, jnp.bitwise_xor(lanes, m))
            return x

        def fire_linear(c):
            bb = c % 2
            _, s, d = sems[bb]
            rb = base + c * CHUNK
            return (
                pltpu.async_copy(src_hbm.at[pl.ds(rb, CHUNK)],
                                 src_v.at[bb], s),
                pltpu.async_copy(dst_hbm.at[pl.ds(rb, CHUNK)],
                                 dst_v.at[bb], d),
            )

        def fire_gather(c):
            bb = c % 2
            return pltpu.async_copy(
                table_sh.at[idx_v.at[pl.ds(c * CHUNK, CHUNK)]],
                rel_v.at[bb], sems[bb][0])

        lin0 = fire_linear(0)
        pltpu.sync_copy(ids_hbm.at[pl.ds(base, B_PER_W)], idx_v)

        @pl.when(sid < 15)
        def _():
            rslab = sid * 64
            pltpu.sync_copy(table_hbm.at[pl.ds(rslab, 64)],
                            table_sh.at[pl.ds(rslab, 64)])

        @pl.when(sid == 15)
        def _():
            pltpu.sync_copy(table_hbm.at[pl.ds(960, 40)],
                            table_sh.at[pl.ds(960, 40)])

        plsc.subcore_barrier()
        inflight = (fire_gather(0),) + lin0

        for c in range(N_CHUNKS):
            bb = c % 2
            nxt = None
            if c + 1 < N_CHUNKS:
                nxt = (fire_gather(c + 1),) + fire_linear(c + 1)
            for h in inflight:
                h.wait()
            inflight = nxt

            def group_body(g, _):
                r0 = g * L

                def row_body(i, res):
                    r = r0 + i
                    acc0 = acc1 = None
                    for j in range(DSL):
                        sl = pl.ds(j * L, L)
                        p = (src_v[bb, r, sl]
                             * rel_v[bb, r, sl]
                             * dst_v[bb, r, sl])
                        if j % 2 == 0:
                            acc0 = p if acc0 is None else acc0 + p
                        else:
                            acc1 = p if acc1 is None else acc1 + p
                    tot = lane_sum(acc0 + acc1)
                    return jnp.where(lanes == i, tot, res)

                res = lax.fori_loop(0, L, row_body,
                                    jnp.zeros((L,), jnp.float32))
                out_v[bb, pl.ds(r0, L)] = res
                return 0

            lax.fori_loop(0, GROUPS, group_body, 0)
            pltpu.sync_copy(out_v.at[bb],
                            out_hbm.at[pl.ds(base + c * CHUNK, CHUNK)])

    return k


_scorer = _sc_kernel()


@jax.jit
def kernel(src_emb, rel_ids, dst_emb, rel_emb_table):
    ids = rel_ids.astype(jnp.int32)
    return _scorer(src_emb, ids, dst_emb, rel_emb_table)
